# baseline (speedup 1.0000x reference)
import functools

import jax
import jax.numpy as jnp
from jax import lax
from jax.experimental import pallas as pl
from jax.experimental.pallas import tpu as pltpu
from jax.experimental.pallas import tpu_sc as plsc

_NC = 2
_NS = 16
_NW = _NC * _NS
_CHUNK = 128


@functools.lru_cache(maxsize=None)
def _make_gather(batch: int, seq: int, n_cols: int, col_base: int,
                 vocab: int, embed: int):
    num_idx = batch * n_cols
    assert num_idx % (_NW * _CHUNK) == 0
    n_per_w = num_idx // _NW
    n_ch = n_per_w // _CHUNK
    assert n_cols % n_per_w == 0
    w_per_row = n_cols // n_per_w

    mesh = plsc.VectorSubcoreMesh(core_axis_name="c", subcore_axis_name="s")

    @functools.partial(
        pl.kernel,
        out_type=jax.ShapeDtypeStruct((num_idx, embed), jnp.float32),
        mesh=mesh,
        scratch_types=[
            pltpu.VMEM((n_ch, _CHUNK), jnp.int32),
            pltpu.VMEM((n_per_w, embed), jnp.float32),
            pltpu.SemaphoreType.DMA,
        ],
    )
    def gather_kernel(seq_hbm, table_hbm, out_hbm, idx_v, rows_v, sem):
        wid = lax.axis_index("s") * _NC + lax.axis_index("c")
        row = wid // w_per_row
        col0 = col_base + (wid % w_per_row) * n_per_w
        for j in range(n_ch):
            pltpu.sync_copy(
                seq_hbm.at[row, pl.ds(col0 + j * _CHUNK, _CHUNK)],
                idx_v.at[j],
            )
        copies = [
            pltpu.async_copy(
                table_hbm.at[idx_v.at[j]],
                rows_v.at[pl.ds(j * _CHUNK, _CHUNK)],
                sem,
            )
            for j in range(n_ch)
        ]
        for c in copies:
            c.wait()
        pltpu.sync_copy(rows_v, out_hbm.at[pl.ds(wid * n_per_w, n_per_w)])

    return gather_kernel


@functools.lru_cache(maxsize=None)
def _make_dense_half(batch: int, seq: int, seq_half: int, embed: int,
                     hidden: int, t_blk: int, off_blk: int, aliased: bool):
    assert seq_half % t_blk == 0
    sblk = seq_half // t_blk
    grid = (sblk, batch)

    def compute(x_ref, w_ref, b_ref, p_ref, g_ref, be_ref, o_ref):
        x = x_ref[...]
        y = jnp.dot(x, w_ref[...], preferred_element_type=jnp.float32)
        y = y + b_ref[...] + p_ref[...]
        mean = jnp.mean(y, axis=-1, keepdims=True)
        yc = y - mean
        var = jnp.mean(yc * yc, axis=-1, keepdims=True)
        o_ref[0] = (g_ref[...] * lax.rsqrt(var + 1e-6)) * yc + be_ref[...]

    if aliased:
        def body(prev_ref, x_ref, w_ref, b_ref, p_ref, g_ref, be_ref, o_ref):
            del prev_ref
            compute(x_ref, w_ref, b_ref, p_ref, g_ref, be_ref, o_ref)
    else:
        body = compute

    in_specs = [
        pl.BlockSpec((t_blk, embed), lambda j, i: (i * sblk + j, 0)),
        pl.BlockSpec((embed, hidden), lambda j, i: (0, 0)),
        pl.BlockSpec((hidden,), lambda j, i: (0,)),
        pl.BlockSpec((t_blk, hidden), lambda j, i: (j + off_blk, 0)),
        pl.BlockSpec((hidden,), lambda j, i: (0,)),
        pl.BlockSpec((hidden,), lambda j, i: (0,)),
    ]
    kwargs = {}
    if aliased:
        in_specs = [pl.BlockSpec(memory_space=pl.ANY)] + in_specs
        kwargs["input_output_aliases"] = {0: 0}

    return pl.pallas_call(
        body,
        grid=grid,
        in_specs=in_specs,
        out_specs=pl.BlockSpec((1, t_blk, hidden),
                               lambda j, i: (i, j + off_blk, 0)),
        out_shape=jax.ShapeDtypeStruct((batch, seq, hidden), jnp.float32),
        **kwargs,
    )


def kernel(sequence, token_table, W, b, pos_table, gamma, beta):
    batch, seq = sequence.shape
    vocab, embed = token_table.shape
    hidden = W.shape[1]
    half = seq // 2
    t_blk = 2048
    nblk_half = half // t_blk

    seq32 = sequence.astype(jnp.int32)
    g_a = _make_gather(batch, seq, half, 0, vocab, embed)(seq32, token_table)
    g_b = _make_gather(batch, seq, half, half, vocab, embed)(seq32, token_table)

    d1 = _make_dense_half(batch, seq, half, embed, hidden, t_blk, 0, False)
    d2 = _make_dense_half(batch, seq, half, embed, hidden, t_blk,
                          nblk_half, True)
    o1 = d1(g_a, W, b, pos_table[:seq], gamma, beta)
    return d2(o1, g_b, W, b, pos_table[:seq], gamma, beta)


# speedup vs baseline: 1.0234x; 1.0234x over previous
import functools

import jax
import jax.numpy as jnp
from jax import lax
from jax.experimental import pallas as pl
from jax.experimental.pallas import tpu as pltpu
from jax.experimental.pallas import tpu_sc as plsc

_NC = 2
_NS = 16
_NW = _NC * _NS
_CHUNK = 128


@functools.lru_cache(maxsize=None)
def _make_gather(batch: int, seq: int, n_cols: int, col_base: int,
                 vocab: int, embed: int):
    num_idx = batch * n_cols
    assert num_idx % (_NW * _CHUNK) == 0
    n_per_w = num_idx // _NW
    n_ch = n_per_w // _CHUNK
    assert n_cols % n_per_w == 0
    w_per_row = n_cols // n_per_w

    mesh = plsc.VectorSubcoreMesh(core_axis_name="c", subcore_axis_name="s")

    @functools.partial(
        pl.kernel,
        out_type=jax.ShapeDtypeStruct((num_idx, embed), jnp.float32),
        mesh=mesh,
        scratch_types=[
            pltpu.VMEM((n_ch, _CHUNK), jnp.int32),
            pltpu.VMEM((n_per_w, embed), jnp.float32),
            pltpu.SemaphoreType.DMA,
        ],
    )
    def gather_kernel(seq_hbm, table_hbm, out_hbm, idx_v, rows_v, sem):
        wid = lax.axis_index("s") * _NC + lax.axis_index("c")
        row = wid // w_per_row
        col0 = col_base + (wid % w_per_row) * n_per_w
        for j in range(n_ch):
            pltpu.sync_copy(
                seq_hbm.at[row, pl.ds(col0 + j * _CHUNK, _CHUNK)],
                idx_v.at[j],
            )
        copies = [
            pltpu.async_copy(
                table_hbm.at[idx_v.at[j]],
                rows_v.at[pl.ds(j * _CHUNK, _CHUNK)],
                sem,
            )
            for j in range(n_ch)
        ]
        for c in copies:
            c.wait()
        pltpu.sync_copy(rows_v, out_hbm.at[pl.ds(wid * n_per_w, n_per_w)])

    return gather_kernel


@functools.lru_cache(maxsize=None)
def _make_dense_half(batch: int, seq: int, seq_half: int, embed: int,
                     hidden: int, t_blk: int, off_blk: int, aliased: bool):
    assert seq_half % t_blk == 0
    sblk = seq_half // t_blk
    grid = (sblk, batch)
    inv_h = 1.0 / hidden

    def compute(x_ref, w_ref, b_ref, p_ref, g_ref, be_ref, o_ref, pb_ref):
        @pl.when(pl.program_id(1) == 0)
        def _():
            pb_ref[...] = b_ref[...] + p_ref[...]

        y = jnp.dot(x_ref[...], w_ref[...],
                    preferred_element_type=jnp.float32)
        t = y + pb_ref[...]
        s1 = jnp.sum(t, axis=-1, keepdims=True)
        s2 = jnp.sum(t * t, axis=-1, keepdims=True)
        mean = s1 * inv_h
        var = s2 * inv_h - mean * mean
        inv = lax.rsqrt(var + 1e-6)
        o_ref[0] = (t * inv - mean * inv) * g_ref[...] + be_ref[...]

    if aliased:
        def body(prev_ref, x_ref, w_ref, b_ref, p_ref, g_ref, be_ref,
                 o_ref, pb_ref):
            del prev_ref
            compute(x_ref, w_ref, b_ref, p_ref, g_ref, be_ref, o_ref, pb_ref)
    else:
        body = compute

    in_specs = [
        pl.BlockSpec((t_blk, embed), lambda j, i: (i * sblk + j, 0)),
        pl.BlockSpec((embed, hidden), lambda j, i: (0, 0)),
        pl.BlockSpec((hidden,), lambda j, i: (0,)),
        pl.BlockSpec((t_blk, hidden), lambda j, i: (j + off_blk, 0)),
        pl.BlockSpec((hidden,), lambda j, i: (0,)),
        pl.BlockSpec((hidden,), lambda j, i: (0,)),
    ]
    kwargs = {}
    if aliased:
        in_specs = [pl.BlockSpec(memory_space=pl.ANY)] + in_specs
        kwargs["input_output_aliases"] = {0: 0}

    return pl.pallas_call(
        body,
        grid=grid,
        in_specs=in_specs,
        out_specs=pl.BlockSpec((1, t_blk, hidden),
                               lambda j, i: (i, j + off_blk, 0)),
        out_shape=jax.ShapeDtypeStruct((batch, seq, hidden), jnp.float32),
        scratch_shapes=[pltpu.VMEM((t_blk, hidden), jnp.float32)],
        **kwargs,
    )


def kernel(sequence, token_table, W, b, pos_table, gamma, beta):
    batch, seq = sequence.shape
    vocab, embed = token_table.shape
    hidden = W.shape[1]
    half = seq // 2
    t_blk = 2048
    nblk_half = half // t_blk

    seq32 = sequence.astype(jnp.int32)
    g_a = _make_gather(batch, seq, half, 0, vocab, embed)(seq32, token_table)
    g_b = _make_gather(batch, seq, half, half, vocab, embed)(seq32, token_table)

    d1 = _make_dense_half(batch, seq, half, embed, hidden, t_blk, 0, False)
    d2 = _make_dense_half(batch, seq, half, embed, hidden, t_blk,
                          nblk_half, True)
    o1 = d1(g_a, W, b, pos_table[:seq], gamma, beta)
    return d2(o1, g_b, W, b, pos_table[:seq], gamma, beta)


# speedup vs baseline: 1.0249x; 1.0015x over previous
import functools

import jax
import jax.numpy as jnp
from jax import lax
from jax.experimental import pallas as pl
from jax.experimental.pallas import tpu as pltpu
from jax.experimental.pallas import tpu_sc as plsc

_NC = 2
_NS = 16
_NW = _NC * _NS
_CHUNK = 128


@functools.lru_cache(maxsize=None)
def _make_gather(batch: int, seq: int, n_cols: int, col_base: int,
                 vocab: int, embed: int):
    num_idx = batch * n_cols
    assert num_idx % (_NW * _CHUNK) == 0
    n_per_w = num_idx // _NW
    n_ch = n_per_w // _CHUNK
    assert n_cols % n_per_w == 0
    w_per_row = n_cols // n_per_w

    mesh = plsc.VectorSubcoreMesh(core_axis_name="c", subcore_axis_name="s")

    @functools.partial(
        pl.kernel,
        out_type=jax.ShapeDtypeStruct((num_idx, embed), jnp.float32),
        mesh=mesh,
        scratch_types=[
            pltpu.VMEM((n_ch, _CHUNK), jnp.int32),
            pltpu.VMEM((n_per_w, embed), jnp.float32),
            pltpu.SemaphoreType.DMA,
        ],
    )
    def gather_kernel(seq_hbm, table_hbm, out_hbm, idx_v, rows_v, sem):
        wid = lax.axis_index("s") * _NC + lax.axis_index("c")
        row = wid // w_per_row
        col0 = col_base + (wid % w_per_row) * n_per_w
        for j in range(n_ch):
            pltpu.sync_copy(
                seq_hbm.at[row, pl.ds(col0 + j * _CHUNK, _CHUNK)],
                idx_v.at[j],
            )
        copies = [
            pltpu.async_copy(
                table_hbm.at[idx_v.at[j]],
                rows_v.at[pl.ds(j * _CHUNK, _CHUNK)],
                sem,
            )
            for j in range(n_ch)
        ]
        for c in copies:
            c.wait()
        pltpu.sync_copy(rows_v, out_hbm.at[pl.ds(wid * n_per_w, n_per_w)])

    return gather_kernel


@functools.lru_cache(maxsize=None)
def _make_dense_half(batch: int, seq: int, seq_half: int, embed: int,
                     hidden: int, t_blk: int, off_blk: int, aliased: bool):
    assert seq_half % t_blk == 0
    sblk = seq_half // t_blk
    grid = (sblk, batch)
    inv_h = 1.0 / hidden

    def compute(x_ref, w_ref, b_ref, p_ref, g_ref, be_ref, o_ref, pb_ref):
        @pl.when(pl.program_id(1) == 0)
        def _():
            pb_ref[...] = b_ref[...] + p_ref[...]

        y = jnp.dot(x_ref[...], w_ref[...],
                    preferred_element_type=jnp.float32)
        t = y + pb_ref[...]
        s1 = jnp.sum(t, axis=-1, keepdims=True)
        s2 = jnp.sum(t * t, axis=-1, keepdims=True)
        mean = s1 * inv_h
        var = s2 * inv_h - mean * mean
        inv = lax.rsqrt(var + 1e-6)
        o_ref[0] = (t * inv - mean * inv) * g_ref[...] + be_ref[...]

    if aliased:
        def body(prev_ref, x_ref, w_ref, b_ref, p_ref, g_ref, be_ref,
                 o_ref, pb_ref):
            del prev_ref
            compute(x_ref, w_ref, b_ref, p_ref, g_ref, be_ref, o_ref, pb_ref)
    else:
        body = compute

    in_specs = [
        pl.BlockSpec((t_blk, embed), lambda j, i: (i * sblk + j, 0)),
        pl.BlockSpec((embed, hidden), lambda j, i: (0, 0)),
        pl.BlockSpec((hidden,), lambda j, i: (0,)),
        pl.BlockSpec((t_blk, hidden), lambda j, i: (j + off_blk, 0)),
        pl.BlockSpec((hidden,), lambda j, i: (0,)),
        pl.BlockSpec((hidden,), lambda j, i: (0,)),
    ]
    kwargs = {}
    if aliased:
        in_specs = [pl.BlockSpec(memory_space=pl.ANY)] + in_specs
        kwargs["input_output_aliases"] = {0: 0}

    return pl.pallas_call(
        body,
        grid=grid,
        in_specs=in_specs,
        out_specs=pl.BlockSpec((1, t_blk, hidden),
                               lambda j, i: (i, j + off_blk, 0)),
        out_shape=jax.ShapeDtypeStruct((batch, seq, hidden), jnp.float32),
        scratch_shapes=[pltpu.VMEM((t_blk, hidden), jnp.float32)],
        **kwargs,
    )


def kernel(sequence, token_table, W, b, pos_table, gamma, beta):
    batch, seq = sequence.shape
    vocab, embed = token_table.shape
    hidden = W.shape[1]
    t_blk = 2048

    seq32 = sequence.astype(jnp.int32)
    g_all = _make_gather(batch, seq, seq, 0, vocab, embed)(seq32, token_table)

    d = _make_dense_half(batch, seq, seq, embed, hidden, t_blk, 0, False)
    return d(g_all, W, b, pos_table[:seq], gamma, beta)
